# baseline (speedup 1.0000x reference)
import functools

import jax
import jax.numpy as jnp
from jax import lax
from jax.experimental import pallas as pl
from jax.experimental.pallas import tpu as pltpu
from jax.experimental.pallas import tpu_sc as plsc

_V, _D = 1_000_000, 64
_B, _C, _K = 16384, 20, 20
_NC, _NS = 2, 16
_NW = _NC * _NS
_BPW = _B // _NW
_NB = 16
_NCHUNK = _BPW // _NB
_ROWS = _NB * _C
_NSTREAM = 4
_SROWS = _ROWS // _NSTREAM
_L = 16
_W = 2 * _D


def _sc_body(idxu_hbm, idxn_hbm, idxt_hbm, paru_hbm, parn_hbm, part_hbm,
             emb_u_hbm, emb_v_hbm,
             pos_hbm, neg_hbm,
             idx_u, idx_n, idx_t, par_u, par_n, par_t,
             rows_u, rows_n, rows_t,
             pos_buf, neg_buf, sem):
    wid = lax.axis_index("s") * _NC + lax.axis_index("c")
    base = wid * _BPW
    pltpu.sync_copy(idxu_hbm.at[pl.ds(base * _C, _BPW * _C)],
                    idx_u.at[pl.ds(0, _BPW * _C)])
    pltpu.sync_copy(idxn_hbm.at[pl.ds(base * _K, _BPW * _K)],
                    idx_n.at[pl.ds(0, _BPW * _K)])
    pltpu.sync_copy(idxt_hbm.at[pl.ds(base, _BPW)], idx_t)
    pltpu.sync_copy(paru_hbm.at[pl.ds(base * _C, _BPW * _C)],
                    par_u.at[pl.ds(0, _BPW * _C)])
    pltpu.sync_copy(parn_hbm.at[pl.ds(base * _K, _BPW * _K)],
                    par_n.at[pl.ds(0, _BPW * _K)])
    pltpu.sync_copy(part_hbm.at[pl.ds(base, _BPW)], par_t)

    lanes = lax.iota(jnp.int32, _L)

    def _lane_sum(x):
        for s in (8, 4, 2, 1):
            x = x + x.at[(lanes + s) % _L].get(mode="promise_in_bounds")
        return x

    def chunk_body(ci, _):
        off = ci * _ROWS
        cps = []
        for s in range(_NSTREAM):
            cps.append(pltpu.async_copy(
                emb_u_hbm.at[idx_u.at[pl.ds(off + s * _SROWS, _SROWS)]],
                rows_u.at[pl.ds(s * _SROWS, _SROWS), :], sem))
            cps.append(pltpu.async_copy(
                emb_v_hbm.at[idx_n.at[pl.ds(off + s * _SROWS, _SROWS)]],
                rows_n.at[pl.ds(s * _SROWS, _SROWS), :], sem))
        cps.append(pltpu.async_copy(
            emb_v_hbm.at[idx_t.at[pl.ds(ci * _NB, _NB)]], rows_t, sem))
        for cp in cps:
            cp.wait()

        def b_body(j, carry):
            acc_p, acc_n = carry
            rb = j * _C
            pu_a = par_u[pl.ds(off + rb, _L)]
            pu_b = par_u[pl.ds(off + rb + _L, _L)]
            pn_a = par_n[pl.ds(off + rb, _L)]
            pn_b = par_n[pl.ds(off + rb + _L, _L)]
            su = [jnp.zeros((_L,), jnp.float32) for _ in range(_D // _L)]
            sn = [jnp.zeros((_L,), jnp.float32) for _ in range(_D // _L)]
            for c in range(_C):
                pu = pu_a[c] if c < _L else pu_b[c - _L]
                pn = pn_a[c] if c < _L else pn_b[c - _L]
                for blk in range(_D // _L):
                    su[blk] = su[blk] + rows_u[rb + c, pl.ds(pu + blk * _L, _L)]
                    sn[blk] = sn[blk] + rows_n[rb + c, pl.ds(pn + blk * _L, _L)]
            pt_vec = par_t[pl.ds(ci * _NB, _L)]
            pt = pt_vec[0] * 0
            for jj in range(_L):
                pt = jnp.where(j == jj, pt_vec[jj], pt)
            pp = jnp.zeros((_L,), jnp.float32)
            nn = jnp.zeros((_L,), jnp.float32)
            for blk in range(_D // _L):
                t_blk = rows_t[j, pl.ds(pt + blk * _L, _L)]
                pp = pp + t_blk * su[blk]
                nn = nn + sn[blk] * su[blk]
            m = lanes == j
            acc_p = jnp.where(m, _lane_sum(pp) * (1.0 / _C), acc_p)
            acc_n = jnp.where(m, _lane_sum(nn) * (1.0 / _C), acc_n)
            return acc_p, acc_n

        acc_p, acc_n = lax.fori_loop(
            0, _NB, b_body,
            (jnp.zeros((_L,), jnp.float32), jnp.zeros((_L,), jnp.float32)))
        pos_buf[pl.ds(ci * _NB, _NB)] = acc_p
        neg_buf[pl.ds(ci * _NB, _NB)] = acc_n
        return 0

    lax.fori_loop(0, _NCHUNK, chunk_body, 0)
    pltpu.sync_copy(pos_buf, pos_hbm.at[pl.ds(base, _BPW)])
    pltpu.sync_copy(neg_buf, neg_hbm.at[pl.ds(base, _BPW)])


@functools.lru_cache(maxsize=None)
def _sc_scores():
    return functools.partial(
        pl.kernel,
        mesh=plsc.VectorSubcoreMesh(core_axis_name="c", subcore_axis_name="s"),
        compiler_params=pltpu.CompilerParams(use_tc_tiling_on_sc=True),
        out_type=[jax.ShapeDtypeStruct((_B,), jnp.float32),
                  jax.ShapeDtypeStruct((_B,), jnp.float32)],
        scratch_types=[
            pltpu.VMEM((_BPW * _C + _L,), jnp.int32),
            pltpu.VMEM((_BPW * _K + _L,), jnp.int32),
            pltpu.VMEM((_BPW,), jnp.int32),
            pltpu.VMEM((_BPW * _C + _L,), jnp.int32),
            pltpu.VMEM((_BPW * _K + _L,), jnp.int32),
            pltpu.VMEM((_BPW,), jnp.int32),
            pltpu.VMEM((_ROWS, _W), jnp.float32),
            pltpu.VMEM((_ROWS, _W), jnp.float32),
            pltpu.VMEM((_NB, _W), jnp.float32),
            pltpu.VMEM((_BPW,), jnp.float32),
            pltpu.VMEM((_BPW,), jnp.float32),
            pltpu.SemaphoreType.DMA,
        ],
    )(_sc_body)


def _finish_body(pos_ref, neg_ref, out_ref):
    p = pos_ref[...]
    n = neg_ref[...]
    lsp = jnp.minimum(p, 0.0) - jnp.log1p(jnp.exp(-jnp.abs(p)))
    lsn = jnp.minimum(-n, 0.0) - jnp.log1p(jnp.exp(-jnp.abs(n)))
    out_ref[0, 0] = -(jnp.sum(lsp) + jnp.sum(lsn)) / _B


def _finish(pos2d, neg2d):
    return pl.pallas_call(
        _finish_body,
        out_shape=jax.ShapeDtypeStruct((1, 1), jnp.float32),
        out_specs=pl.BlockSpec(memory_space=pltpu.SMEM),
    )(pos2d, neg2d)


def kernel(inputs, targets, negatives, emb_u, emb_v):
    inputs = inputs.astype(jnp.int32)
    targets = targets.astype(jnp.int32)
    negatives = negatives.astype(jnp.int32)
    idxu = (inputs >> 1).reshape(-1)
    paru = ((inputs & 1) * _D).reshape(-1)
    idxn = (negatives >> 1).reshape(-1)
    parn = ((negatives & 1) * _D).reshape(-1)
    idxt = targets >> 1
    part = (targets & 1) * _D
    emb_u2 = emb_u.reshape(_V // 2, _W)
    emb_v2 = emb_v.reshape(_V // 2, _W)
    pos, neg = _sc_scores()(idxu, idxn, idxt, paru, parn, part,
                            emb_u2, emb_v2)
    res = _finish(pos.reshape(128, 128), neg.reshape(128, 128))
    return res[0, 0]


# speedup vs baseline: 1.1371x; 1.1371x over previous
import functools

import jax
import jax.numpy as jnp
from jax import lax
from jax.experimental import pallas as pl
from jax.experimental.pallas import tpu as pltpu
from jax.experimental.pallas import tpu_sc as plsc

_V, _D = 1_000_000, 64
_B, _C, _K = 16384, 20, 20
_NC, _NS = 2, 16
_NW = _NC * _NS
_BPW = _B // _NW
_NB = 16
_NCHUNK = _BPW // _NB
_ROWS = _NB * _C
_NSTREAM = 4
_SROWS = _ROWS // _NSTREAM
_L = 16


def _sc_body(idxu_hbm, idxn_hbm, idxt_hbm, emb_u_hbm, emb_v_hbm,
             pos_hbm, neg_hbm,
             idx_u, idx_n, idx_t, rows_u, rows_n, rows_t,
             pos_buf, neg_buf, sem):
    wid = lax.axis_index("s") * _NC + lax.axis_index("c")
    base = wid * _BPW
    pltpu.sync_copy(idxu_hbm.at[pl.ds(base * _C, _BPW * _C)], idx_u)
    pltpu.sync_copy(idxn_hbm.at[pl.ds(base * _K, _BPW * _K)], idx_n)
    pltpu.sync_copy(idxt_hbm.at[pl.ds(base, _BPW)], idx_t)

    lanes = lax.iota(jnp.int32, _L)

    def _lane_sum(x):
        for s in (8, 4, 2, 1):
            x = x + x.at[(lanes + s) % _L].get(mode="promise_in_bounds")
        return x

    def chunk_body(ci, _):
        off = ci * _ROWS
        cps = []
        for s in range(_NSTREAM):
            cps.append(pltpu.async_copy(
                emb_u_hbm.at[idx_u.at[pl.ds(off + s * _SROWS, _SROWS)]],
                rows_u.at[pl.ds(s * _SROWS, _SROWS), :], sem))
            cps.append(pltpu.async_copy(
                emb_v_hbm.at[idx_n.at[pl.ds(off + s * _SROWS, _SROWS)]],
                rows_n.at[pl.ds(s * _SROWS, _SROWS), :], sem))
        cps.append(pltpu.async_copy(
            emb_v_hbm.at[idx_t.at[pl.ds(ci * _NB, _NB)]], rows_t, sem))
        for cp in cps:
            cp.wait()

        def b_body(j, carry):
            acc_p, acc_n = carry
            rb = j * _C
            su = [jnp.zeros((_L,), jnp.float32) for _ in range(_D // _L)]
            sn = [jnp.zeros((_L,), jnp.float32) for _ in range(_D // _L)]
            for c in range(_C):
                for blk in range(_D // _L):
                    su[blk] = su[blk] + rows_u[rb + c, pl.ds(blk * _L, _L)]
                    sn[blk] = sn[blk] + rows_n[rb + c, pl.ds(blk * _L, _L)]
            pp = jnp.zeros((_L,), jnp.float32)
            nn = jnp.zeros((_L,), jnp.float32)
            for blk in range(_D // _L):
                t_blk = rows_t[j, pl.ds(blk * _L, _L)]
                pp = pp + t_blk * su[blk]
                nn = nn + sn[blk] * su[blk]
            m = lanes == j
            acc_p = jnp.where(m, _lane_sum(pp) * (1.0 / _C), acc_p)
            acc_n = jnp.where(m, _lane_sum(nn) * (1.0 / _C), acc_n)
            return acc_p, acc_n

        acc_p, acc_n = lax.fori_loop(
            0, _NB, b_body,
            (jnp.zeros((_L,), jnp.float32), jnp.zeros((_L,), jnp.float32)))
        pos_buf[pl.ds(ci * _NB, _NB)] = acc_p
        neg_buf[pl.ds(ci * _NB, _NB)] = acc_n
        return 0

    lax.fori_loop(0, _NCHUNK, chunk_body, 0)
    pltpu.sync_copy(pos_buf, pos_hbm.at[pl.ds(base, _BPW)])
    pltpu.sync_copy(neg_buf, neg_hbm.at[pl.ds(base, _BPW)])


@functools.lru_cache(maxsize=None)
def _sc_scores():
    return functools.partial(
        pl.kernel,
        mesh=plsc.VectorSubcoreMesh(core_axis_name="c", subcore_axis_name="s"),
        compiler_params=pltpu.CompilerParams(use_tc_tiling_on_sc=False),
        out_type=[jax.ShapeDtypeStruct((_B,), jnp.float32),
                  jax.ShapeDtypeStruct((_B,), jnp.float32)],
        scratch_types=[
            pltpu.VMEM((_BPW * _C,), jnp.int32),
            pltpu.VMEM((_BPW * _K,), jnp.int32),
            pltpu.VMEM((_BPW,), jnp.int32),
            pltpu.VMEM((_ROWS, _D), jnp.float32),
            pltpu.VMEM((_ROWS, _D), jnp.float32),
            pltpu.VMEM((_NB, _D), jnp.float32),
            pltpu.VMEM((_BPW,), jnp.float32),
            pltpu.VMEM((_BPW,), jnp.float32),
            pltpu.SemaphoreType.DMA,
        ],
    )(_sc_body)


def _finish_body(pos_ref, neg_ref, out_ref):
    p = pos_ref[...]
    n = neg_ref[...]
    lsp = jnp.minimum(p, 0.0) - jnp.log1p(jnp.exp(-jnp.abs(p)))
    lsn = jnp.minimum(-n, 0.0) - jnp.log1p(jnp.exp(-jnp.abs(n)))
    out_ref[0, 0] = -(jnp.sum(lsp) + jnp.sum(lsn)) / _B


def _finish(pos2d, neg2d):
    return pl.pallas_call(
        _finish_body,
        out_shape=jax.ShapeDtypeStruct((1, 1), jnp.float32),
        out_specs=pl.BlockSpec(memory_space=pltpu.SMEM),
    )(pos2d, neg2d)


def kernel(inputs, targets, negatives, emb_u, emb_v):
    idxu = inputs.astype(jnp.int32).reshape(-1)
    idxn = negatives.astype(jnp.int32).reshape(-1)
    idxt = targets.astype(jnp.int32)
    pos, neg = _sc_scores()(idxu, idxn, idxt, emb_u, emb_v)
    res = _finish(pos.reshape(128, 128), neg.reshape(128, 128))
    return res[0, 0]
